# Initial kernel scaffold; baseline (speedup 1.0000x reference)
import jax
import jax.numpy as jnp
from jax.experimental import pallas as pl


def kernel(q, k, v, attn_mask):
    raise NotImplementedError("write your pallas kernel here")


# speedup vs baseline: 2.6486x; 2.6486x over previous
import functools
import math

import numpy as np
import jax
import jax.numpy as jnp
from jax.experimental import pallas as pl


@functools.lru_cache(maxsize=None)
def _sample_counts(L_q: int, L_k: int, u_part: int):
    with jax.ensure_compile_time_eval():
        idx = np.asarray(
            jax.random.randint(jax.random.key(42), (L_q, u_part), 0, L_k))
    cnt = np.zeros((L_k, L_q), np.float32)
    np.add.at(cnt, (idx, np.arange(L_q)[:, None]), 1.0)
    return cnt


def _probattn_body(cnt_ref, q_ref, k_ref, v_ref, o_ref, *, u, u_pad, tq):
    q = q_ref[0]
    k = k_ref[0]
    v = v_ref[0]
    L_q, D = q.shape
    L_k = k.shape[0]
    hi = jax.lax.Precision.HIGHEST

    m_parts = []
    for t in range(L_q // tq):
        qs = q[t * tq:(t + 1) * tq]
        s = jax.lax.dot_general(k, qs, (((1,), (1,)), ((), ())),
                                precision=hi,
                                preferred_element_type=jnp.float32)
        c = cnt_ref[:, t * tq:(t + 1) * tq]
        smax = jnp.max(jnp.where(c > 0.0, s, -1e30), axis=0, keepdims=True)
        ssum = jnp.sum(s * c, axis=0, keepdims=True)
        m_parts.append(smax - ssum * (1.0 / L_k))
    m_row = jnp.concatenate(m_parts, axis=1)

    col = jax.lax.broadcasted_iota(jnp.int32, (1, L_q), 1)

    def body(i, carry):
        mv, rank = carry
        mx = jnp.max(mv)
        idx = jnp.min(jnp.where(mv == mx, col, L_q))
        hit = col == idx
        rank = jnp.where(hit, i + 1, rank)
        mv = jnp.where(hit, -3e30, mv)
        return mv, rank

    _, rank = jax.lax.fori_loop(
        0, u, body, (m_row, jnp.zeros((1, L_q), jnp.int32)))

    rowi = jax.lax.broadcasted_iota(jnp.int32, (u_pad, L_q), 0)
    oh = (rank == rowi + 1).astype(jnp.float32)

    qr = jax.lax.dot_general(oh, q, (((1,), (0,)), ((), ())), precision=hi,
                             preferred_element_type=jnp.float32)
    sc = jax.lax.dot_general(qr, k, (((1,), (1,)), ((), ())), precision=hi,
                             preferred_element_type=jnp.float32)
    sc = sc * (1.0 / math.sqrt(D))
    sc = sc - jnp.max(sc, axis=1, keepdims=True)
    e = jnp.exp(sc)
    attn = e / jnp.sum(e, axis=1, keepdims=True)
    ctx = jax.lax.dot_general(attn, v, (((1,), (0,)), ((), ())), precision=hi,
                              preferred_element_type=jnp.float32)

    scat = jax.lax.dot_general(oh, ctx, (((0,), (0,)), ((), ())), precision=hi,
                               preferred_element_type=jnp.float32)
    ones_u = jnp.ones((u_pad, 1), jnp.float32)
    selcol = jax.lax.dot_general(oh, ones_u, (((0,), (0,)), ((), ())),
                                 precision=hi,
                                 preferred_element_type=jnp.float32)
    o_ref[0] = v * (1.0 - selcol) + scat


def kernel(q, k, v, attn_mask):
    B, L_q, H, D = q.shape
    L_k = k.shape[1]
    assert L_q == L_k and B == 1
    factor = 5
    u_part = min(factor * int(np.ceil(np.log(L_k))), L_k)
    u = min(factor * int(np.ceil(np.log(L_q))), L_q)
    u_pad = -(-u // 8) * 8
    cnt = jnp.asarray(_sample_counts(L_q, L_k, u_part))

    qt = jnp.transpose(q, (0, 2, 1, 3)).reshape(B * H, L_q, D)
    kt = jnp.transpose(k, (0, 2, 1, 3)).reshape(B * H, L_k, D)
    vt = jnp.transpose(v, (0, 2, 1, 3)).reshape(B * H, L_k, D)

    body = functools.partial(_probattn_body, u=u, u_pad=u_pad, tq=512)
    out = pl.pallas_call(
        body,
        grid=(B * H,),
        in_specs=[
            pl.BlockSpec((L_k, L_q), lambda h: (0, 0)),
            pl.BlockSpec((1, L_q, D), lambda h: (h, 0, 0)),
            pl.BlockSpec((1, L_k, D), lambda h: (h, 0, 0)),
            pl.BlockSpec((1, L_k, D), lambda h: (h, 0, 0)),
        ],
        out_specs=pl.BlockSpec((1, L_k, D), lambda h: (h, 0, 0)),
        out_shape=jax.ShapeDtypeStruct((B * H, L_k, D), jnp.float32),
    )(cnt, qt, kt, vt)
    return out.reshape(B, H, L_k, D).transpose(0, 2, 1, 3)
